# baseline (speedup 1.0000x reference)
import functools

import jax
import jax.numpy as jnp
from jax import lax
from jax.experimental import pallas as pl
from jax.experimental.pallas import tpu as pltpu
from jax.experimental.pallas import tpu_sc as plsc

N = 10000
NPAD = 10240
PADNODE = 10016
E = 320000
NC = 2
NS = 16
NW = NC * NS
C = 128
CPW = 80
EPW = CPW * C
EPAD = EPW * NW
ROWS_PER_SUB = NPAD // NS

_mesh = plsc.VectorSubcoreMesh(core_axis_name="c", subcore_axis_name="s")


@functools.partial(
    pl.kernel,
    mesh=_mesh,
    out_type=jax.ShapeDtypeStruct((NC, NPAD, 128), jnp.float32),
    scratch_types=[
        pltpu.VMEM((C,), jnp.int32),
        pltpu.VMEM((C,), jnp.int32),
        pltpu.VMEM((C,), jnp.int32),
        pltpu.VMEM((C,), jnp.int32),
        pltpu.VMEM((C, 128), jnp.float32),
        pltpu.VMEM((C, 128), jnp.float32),
        pltpu.VMEM_SHARED((NPAD, 128), jnp.float32),
        pltpu.SemaphoreType.DMA,
        pltpu.SemaphoreType.DMA,
        pltpu.SemaphoreType.DMA,
        pltpu.SemaphoreType.DMA,
        pltpu.SemaphoreType.DMA,
        pltpu.SemaphoreType.DMA,
    ],
)
def _edge_scatter(g_hbm, src_hbm, dst_hbm, zeros_hbm, out_hbm,
                  sv0, sv1, dv0, dv1, r0, r1, acc,
                  ssrc0, ssrc1, sdst0, sdst1, sg0, sg1):
  cid = lax.axis_index("c")
  sid = lax.axis_index("s")
  wid = sid * NC + cid
  ebase = wid * EPW
  rbase = sid * ROWS_PER_SUB
  sv = (sv0, sv1)
  dv = (dv0, dv1)
  rows = (r0, r1)
  ssrc = (ssrc0, ssrc1)
  sdst = (sdst0, sdst1)
  sg = (sg0, sg1)

  def _off(t):
    return ebase + jnp.where(t >= CPW, t - CPW, t) * C

  pltpu.async_copy(src_hbm.at[pl.ds(_off(0), C)], sv0, ssrc0)
  pltpu.async_copy(src_hbm.at[pl.ds(_off(1), C)], sv1, ssrc1)
  pltpu.async_copy(dst_hbm.at[pl.ds(_off(0), C)], dv0, sdst0)
  pltpu.async_copy(dst_hbm.at[pl.ds(_off(1), C)], dv1, sdst1)
  pltpu.sync_copy(zeros_hbm.at[pl.ds(rbase, ROWS_PER_SUB)],
                  acc.at[pl.ds(rbase, ROWS_PER_SUB)])
  plsc.subcore_barrier()
  pltpu.make_async_copy(src_hbm.at[pl.ds(0, C)], sv0, ssrc0).wait()
  pltpu.async_copy(g_hbm.at[sv0], r0, sg0)

  def slot(t, p):
    pltpu.make_async_copy(src_hbm.at[pl.ds(0, C)], sv[1 - p], ssrc[1 - p]).wait()
    pltpu.async_copy(g_hbm.at[sv[1 - p]], rows[1 - p], sg[1 - p])
    pltpu.make_async_copy(g_hbm.at[sv[p]], rows[p], sg[p]).wait()
    pltpu.async_copy(src_hbm.at[pl.ds(_off(t + 2), C)], sv[p], ssrc[p])
    pltpu.make_async_copy(dst_hbm.at[pl.ds(0, C)], dv[p], sdst[p]).wait()
    pltpu.sync_copy(rows[p], acc.at[dv[p]], add=True)
    pltpu.async_copy(dst_hbm.at[pl.ds(_off(t + 2), C)], dv[p], sdst[p])

  def body(i, carry):
    slot(2 * i, 0)
    slot(2 * i + 1, 1)
    return carry

  lax.fori_loop(0, CPW // 2, body, 0)

  pltpu.make_async_copy(src_hbm.at[pl.ds(0, C)], sv1, ssrc1).wait()
  pltpu.async_copy(g_hbm.at[sv1], r1, sg1)
  pltpu.make_async_copy(g_hbm.at[sv0], r0, sg0).wait()
  pltpu.make_async_copy(g_hbm.at[sv1], r1, sg1).wait()
  pltpu.make_async_copy(dst_hbm.at[pl.ds(0, C)], dv0, sdst0).wait()
  pltpu.make_async_copy(dst_hbm.at[pl.ds(0, C)], dv1, sdst1).wait()
  plsc.subcore_barrier()
  pltpu.sync_copy(acc.at[pl.ds(rbase, ROWS_PER_SUB)],
                  out_hbm.at[cid].at[pl.ds(rbase, ROWS_PER_SUB)])


DEGW = 128


@functools.partial(
    pl.kernel,
    mesh=_mesh,
    out_type=jax.ShapeDtypeStruct((NC, NPAD, DEGW), jnp.float32),
    scratch_types=[
        pltpu.VMEM((C,), jnp.int32),
        pltpu.VMEM((C,), jnp.int32),
        pltpu.VMEM((C, DEGW), jnp.float32),
        pltpu.VMEM_SHARED((NPAD, DEGW), jnp.float32),
        pltpu.SemaphoreType.DMA,
        pltpu.SemaphoreType.DMA,
    ],
)
def _deg_kernel(dst_hbm, zeros_hbm, ones_hbm, out_hbm,
                dv0, dv1, ones_v, acc, sd0, sd1):
  cid = lax.axis_index("c")
  sid = lax.axis_index("s")
  wid = sid * NC + cid
  ebase = wid * EPW
  base = sid * ROWS_PER_SUB
  dv = (dv0, dv1)
  sd = (sd0, sd1)

  def _off(t):
    return ebase + jnp.where(t >= CPW, t - CPW, t) * C

  pltpu.async_copy(dst_hbm.at[pl.ds(_off(0), C)], dv0, sd0)
  pltpu.async_copy(dst_hbm.at[pl.ds(_off(1), C)], dv1, sd1)
  pltpu.sync_copy(ones_hbm, ones_v)
  pltpu.sync_copy(zeros_hbm.at[pl.ds(base, ROWS_PER_SUB)],
                  acc.at[pl.ds(base, ROWS_PER_SUB)])
  plsc.subcore_barrier()

  def slot(t, p):
    pltpu.make_async_copy(dst_hbm.at[pl.ds(0, C)], dv[p], sd[p]).wait()
    pltpu.sync_copy(ones_v, acc.at[dv[p]], add=True)
    pltpu.async_copy(dst_hbm.at[pl.ds(_off(t + 2), C)], dv[p], sd[p])

  def body(i, carry):
    slot(2 * i, 0)
    slot(2 * i + 1, 1)
    return carry

  lax.fori_loop(0, CPW // 2, body, 0)
  pltpu.make_async_copy(dst_hbm.at[pl.ds(0, C)], dv0, sd0).wait()
  pltpu.make_async_copy(dst_hbm.at[pl.ds(0, C)], dv1, sd1).wait()
  plsc.subcore_barrier()
  pltpu.sync_copy(acc.at[pl.ds(base, ROWS_PER_SUB)],
                  out_hbm.at[cid].at[pl.ds(base, ROWS_PER_SUB)])


R = 1024
GRID = NPAD // R


def _dinv_body(degp_ref, o_ref):
  deg = degp_ref[0, :, 0] + degp_ref[1, :, 0] + 1.0
  o_ref[...] = lax.rsqrt(deg)[:, None]


def _mm1_body(x_ref, w_ref, dinv_ref, o_ref):
  h = jnp.dot(x_ref[...], w_ref[...], preferred_element_type=jnp.float32)
  o_ref[...] = h * dinv_ref[...]


def _mm2_body(s_ref, g1_ref, dinv_ref, b1_ref, w2_ref, o_ref):
  dinv = dinv_ref[...]
  a = (s_ref[0] + s_ref[1] + g1_ref[...]) * dinv + b1_ref[...]
  a = jnp.maximum(a, 0.0)
  h = jnp.dot(a, w2_ref[...], preferred_element_type=jnp.float32)
  o_ref[...] = jnp.concatenate(
      [h * dinv, jnp.zeros((R, 64), jnp.float32)], axis=1)


def _fin_body(s_ref, g2_ref, dinv_ref, b2_ref, o_ref):
  z = ((s_ref[0, :, :64] + s_ref[1, :, :64] + g2_ref[:, :64])
       * dinv_ref[...] + b2_ref[...])
  m = jnp.max(z, axis=1, keepdims=True)
  zs = z - m
  o_ref[...] = zs - jnp.log(jnp.sum(jnp.exp(zs), axis=1, keepdims=True))


def _row_spec(width):
  return pl.BlockSpec((R, width), lambda i: (i, 0))


def _pair_spec(width):
  return pl.BlockSpec((NC, R, width), lambda i: (0, i, 0))


_dinv_spec = pl.BlockSpec((R, 1), lambda i: (i, 0))
_full = lambda shape: pl.BlockSpec(shape, lambda i: (0,) * len(shape))


def _dinv_call(degp):
  return pl.pallas_call(
      _dinv_body,
      grid=(1,),
      in_specs=[pl.BlockSpec((NC, NPAD, DEGW), lambda i: (0, 0, 0))],
      out_specs=pl.BlockSpec((NPAD, 1), lambda i: (0, 0)),
      out_shape=jax.ShapeDtypeStruct((NPAD, 1), jnp.float32),
  )(degp)


def _mm1_call(x, W1, dinv):
  return pl.pallas_call(
      _mm1_body,
      grid=(GRID,),
      in_specs=[_row_spec(128), _full((128, 128)), _dinv_spec],
      out_specs=_row_spec(128),
      out_shape=jax.ShapeDtypeStruct((NPAD, 128), jnp.float32),
  )(x, W1, dinv)


def _mm2_call(s1, g1, dinv, b1, W2):
  return pl.pallas_call(
      _mm2_body,
      grid=(GRID,),
      in_specs=[_pair_spec(128), _row_spec(128), _dinv_spec,
                _full((1, 128)), _full((128, 64))],
      out_specs=_row_spec(128),
      out_shape=jax.ShapeDtypeStruct((NPAD, 128), jnp.float32),
  )(s1, g1, dinv, b1, W2)


def _fin_call(s2, g2, dinv, b2):
  return pl.pallas_call(
      _fin_body,
      grid=(GRID,),
      in_specs=[_pair_spec(128), _row_spec(128), _dinv_spec, _full((1, 64))],
      out_specs=_row_spec(64),
      out_shape=jax.ShapeDtypeStruct((N, 64), jnp.float32),
  )(s2, g2, dinv, b2)


@jax.jit
def kernel(x, edge_index, W1, b1, W2, b2):
  ei = edge_index.astype(jnp.int32)
  pad = jnp.full((EPAD - E,), PADNODE, jnp.int32)
  src = jnp.concatenate([ei[0], pad])
  dst = jnp.concatenate([ei[1], pad])
  xp = jnp.pad(x, ((0, NPAD - N), (0, 0)))
  zeros128 = jnp.zeros((NPAD, 128), jnp.float32)
  ones = jnp.ones((C, DEGW), jnp.float32)

  degp = _deg_kernel(dst, zeros128, ones)
  dinv = _dinv_call(degp)
  g1 = _mm1_call(xp, W1, dinv)
  s1 = _edge_scatter(g1, src, dst, zeros128)
  g2 = _mm2_call(s1, g1, dinv, b1.reshape(1, -1), W2)
  s2 = _edge_scatter(g2, src, dst, zeros128)
  return _fin_call(s2, g2, dinv, b2.reshape(1, -1))


# speedup vs baseline: 1.0894x; 1.0894x over previous
import functools

import jax
import jax.numpy as jnp
from jax import lax
from jax.experimental import pallas as pl
from jax.experimental.pallas import tpu as pltpu
from jax.experimental.pallas import tpu_sc as plsc

N = 10000
NPAD = 10240
PADNODE = 10016
E = 320000
NC = 2
NS = 16
NW = NC * NS
C = 128
CPW = 80
EPW = CPW * C
EPAD = EPW * NW
ROWS_PER_SUB = NPAD // NS

_mesh = plsc.VectorSubcoreMesh(core_axis_name="c", subcore_axis_name="s")


@functools.partial(
    pl.kernel,
    mesh=_mesh,
    out_type=jax.ShapeDtypeStruct((NC, NPAD, 128), jnp.float32),
    scratch_types=[
        pltpu.VMEM((C,), jnp.int32),
        pltpu.VMEM((C,), jnp.int32),
        pltpu.VMEM((C,), jnp.int32),
        pltpu.VMEM((C,), jnp.int32),
        pltpu.VMEM((C, 128), jnp.float32),
        pltpu.VMEM((C, 128), jnp.float32),
        pltpu.VMEM_SHARED((NPAD, 128), jnp.float32),
        pltpu.SemaphoreType.DMA,
        pltpu.SemaphoreType.DMA,
        pltpu.SemaphoreType.DMA,
        pltpu.SemaphoreType.DMA,
        pltpu.SemaphoreType.DMA,
        pltpu.SemaphoreType.DMA,
    ],
)
def _edge_scatter(g_hbm, src_hbm, dst_hbm, zeros_hbm, out_hbm,
                  sv0, sv1, dv0, dv1, r0, r1, acc,
                  ssrc0, ssrc1, sdst0, sdst1, sg0, sg1):
  cid = lax.axis_index("c")
  sid = lax.axis_index("s")
  wid = sid * NC + cid
  ebase = wid * EPW
  rbase = sid * ROWS_PER_SUB
  sv = (sv0, sv1)
  dv = (dv0, dv1)
  rows = (r0, r1)
  ssrc = (ssrc0, ssrc1)
  sdst = (sdst0, sdst1)
  sg = (sg0, sg1)

  def _off(t):
    return ebase + jnp.where(t >= CPW, t - CPW, t) * C

  pltpu.async_copy(src_hbm.at[pl.ds(_off(0), C)], sv0, ssrc0)
  pltpu.async_copy(src_hbm.at[pl.ds(_off(1), C)], sv1, ssrc1)
  pltpu.async_copy(dst_hbm.at[pl.ds(_off(0), C)], dv0, sdst0)
  pltpu.async_copy(dst_hbm.at[pl.ds(_off(1), C)], dv1, sdst1)
  pltpu.sync_copy(zeros_hbm.at[pl.ds(rbase, ROWS_PER_SUB)],
                  acc.at[pl.ds(rbase, ROWS_PER_SUB)])
  plsc.subcore_barrier()
  pltpu.make_async_copy(src_hbm.at[pl.ds(0, C)], sv0, ssrc0).wait()
  pltpu.async_copy(g_hbm.at[sv0], r0, sg0)

  def slot(t, p):
    pltpu.make_async_copy(src_hbm.at[pl.ds(0, C)], sv[1 - p], ssrc[1 - p]).wait()
    pltpu.async_copy(g_hbm.at[sv[1 - p]], rows[1 - p], sg[1 - p])
    pltpu.make_async_copy(g_hbm.at[sv[p]], rows[p], sg[p]).wait()
    pltpu.async_copy(src_hbm.at[pl.ds(_off(t + 2), C)], sv[p], ssrc[p])
    pltpu.make_async_copy(dst_hbm.at[pl.ds(0, C)], dv[p], sdst[p]).wait()
    pltpu.sync_copy(rows[p], acc.at[dv[p]], add=True)
    pltpu.async_copy(dst_hbm.at[pl.ds(_off(t + 2), C)], dv[p], sdst[p])

  def body(i, carry):
    slot(2 * i, 0)
    slot(2 * i + 1, 1)
    return carry

  lax.fori_loop(0, CPW // 2, body, 0)

  pltpu.make_async_copy(src_hbm.at[pl.ds(0, C)], sv1, ssrc1).wait()
  pltpu.async_copy(g_hbm.at[sv1], r1, sg1)
  pltpu.make_async_copy(g_hbm.at[sv0], r0, sg0).wait()
  pltpu.make_async_copy(g_hbm.at[sv1], r1, sg1).wait()
  pltpu.make_async_copy(dst_hbm.at[pl.ds(0, C)], dv0, sdst0).wait()
  pltpu.make_async_copy(dst_hbm.at[pl.ds(0, C)], dv1, sdst1).wait()
  plsc.subcore_barrier()
  pltpu.sync_copy(acc.at[pl.ds(rbase, ROWS_PER_SUB)],
                  out_hbm.at[cid].at[pl.ds(rbase, ROWS_PER_SUB)])


DEGW = 128


@functools.partial(
    pl.kernel,
    mesh=_mesh,
    out_type=jax.ShapeDtypeStruct((NC, NPAD, DEGW), jnp.float32),
    scratch_types=[
        pltpu.VMEM((C,), jnp.int32),
        pltpu.VMEM((C,), jnp.int32),
        pltpu.VMEM((C, DEGW), jnp.float32),
        pltpu.VMEM_SHARED((NPAD, DEGW), jnp.float32),
        pltpu.SemaphoreType.DMA,
        pltpu.SemaphoreType.DMA,
    ],
)
def _deg_kernel(dst_hbm, zeros_hbm, ones_hbm, out_hbm,
                dv0, dv1, ones_v, acc, sd0, sd1):
  cid = lax.axis_index("c")
  sid = lax.axis_index("s")
  wid = sid * NC + cid
  ebase = wid * EPW
  base = sid * ROWS_PER_SUB
  dv = (dv0, dv1)
  sd = (sd0, sd1)

  def _off(t):
    return ebase + jnp.where(t >= CPW, t - CPW, t) * C

  pltpu.async_copy(dst_hbm.at[pl.ds(_off(0), C)], dv0, sd0)
  pltpu.async_copy(dst_hbm.at[pl.ds(_off(1), C)], dv1, sd1)
  pltpu.sync_copy(ones_hbm, ones_v)
  pltpu.sync_copy(zeros_hbm.at[pl.ds(base, ROWS_PER_SUB)],
                  acc.at[pl.ds(base, ROWS_PER_SUB)])
  plsc.subcore_barrier()

  def slot(t, p):
    pltpu.make_async_copy(dst_hbm.at[pl.ds(0, C)], dv[p], sd[p]).wait()
    pltpu.sync_copy(ones_v, acc.at[dv[p]], add=True)
    pltpu.async_copy(dst_hbm.at[pl.ds(_off(t + 2), C)], dv[p], sd[p])

  def body(i, carry):
    slot(2 * i, 0)
    slot(2 * i + 1, 1)
    return carry

  lax.fori_loop(0, CPW // 2, body, 0)
  pltpu.make_async_copy(dst_hbm.at[pl.ds(0, C)], dv0, sd0).wait()
  pltpu.make_async_copy(dst_hbm.at[pl.ds(0, C)], dv1, sd1).wait()
  plsc.subcore_barrier()
  pltpu.sync_copy(acc.at[pl.ds(base, ROWS_PER_SUB)],
                  out_hbm.at[cid].at[pl.ds(base, ROWS_PER_SUB)])


R = 1024
GRID = NPAD // R


def _dinv_body(degp_ref, o_ref):
  deg = degp_ref[0, :, 0] + degp_ref[1, :, 0] + 1.0
  o_ref[...] = lax.rsqrt(deg)[:, None]


def _mm1_body(x_ref, w_ref, dinv_ref, o_ref):
  h = jnp.dot(x_ref[...], w_ref[...], preferred_element_type=jnp.float32)
  o_ref[...] = h * dinv_ref[...]


def _mm2_body(s_ref, g1_ref, dinv_ref, b1_ref, w2_ref, o_ref):
  dinv = dinv_ref[...]
  a = (s_ref[0] + s_ref[1] + g1_ref[...]) * dinv + b1_ref[...]
  a = jnp.maximum(a, 0.0)
  h = jnp.dot(a, w2_ref[...], preferred_element_type=jnp.float32)
  o_ref[...] = jnp.concatenate(
      [h * dinv, jnp.zeros((R, 64), jnp.float32)], axis=1)


def _fin_body(s_ref, g2_ref, dinv_ref, b2_ref, o_ref):
  z = ((s_ref[0, :, :64] + s_ref[1, :, :64] + g2_ref[:, :64])
       * dinv_ref[...] + b2_ref[...])
  m = jnp.max(z, axis=1, keepdims=True)
  zs = z - m
  o_ref[...] = zs - jnp.log(jnp.sum(jnp.exp(zs), axis=1, keepdims=True))


def _row_spec(width):
  return pl.BlockSpec((R, width), lambda i: (i, 0))


def _pair_spec(width):
  return pl.BlockSpec((NC, R, width), lambda i: (0, i, 0))


_dinv_spec = pl.BlockSpec((R, 1), lambda i: (i, 0))
_full = lambda shape: pl.BlockSpec(shape, lambda i: (0,) * len(shape))


def _dinv_call(degp):
  return pl.pallas_call(
      _dinv_body,
      grid=(1,),
      in_specs=[pl.BlockSpec((NC, NPAD, DEGW), lambda i: (0, 0, 0))],
      out_specs=pl.BlockSpec((NPAD, 1), lambda i: (0, 0)),
      out_shape=jax.ShapeDtypeStruct((NPAD, 1), jnp.float32),
  )(degp)


def _mm1_call(x, W1, dinv):
  return pl.pallas_call(
      _mm1_body,
      grid=(GRID,),
      in_specs=[_row_spec(128), _full((128, 128)), _dinv_spec],
      out_specs=_row_spec(128),
      out_shape=jax.ShapeDtypeStruct((NPAD, 128), jnp.float32),
  )(x, W1, dinv)


def _mm2_call(s1, g1, dinv, b1, W2):
  return pl.pallas_call(
      _mm2_body,
      grid=(GRID,),
      in_specs=[_pair_spec(128), _row_spec(128), _dinv_spec,
                _full((1, 128)), _full((128, 64))],
      out_specs=_row_spec(128),
      out_shape=jax.ShapeDtypeStruct((NPAD, 128), jnp.float32),
  )(s1, g1, dinv, b1, W2)


def _fin_call(s2, g2, dinv, b2):
  return pl.pallas_call(
      _fin_body,
      grid=(GRID,),
      in_specs=[_pair_spec(128), _row_spec(128), _dinv_spec, _full((1, 64))],
      out_specs=_row_spec(64),
      out_shape=jax.ShapeDtypeStruct((N, 64), jnp.float32),
  )(s2, g2, dinv, b2)


@jax.jit
def kernel(x, edge_index, W1, b1, W2, b2):
  ei = edge_index.astype(jnp.int32)
  pad_src = jnp.full((EPAD - E,), PADNODE, jnp.int32)
  pad_dst = N + jnp.arange(EPAD - E, dtype=jnp.int32) % (NPAD - N)
  src = jnp.concatenate([ei[0], pad_src])
  dst = jnp.concatenate([ei[1], pad_dst])
  xp = jnp.pad(x, ((0, NPAD - N), (0, 0)))
  zeros128 = jnp.zeros((NPAD, 128), jnp.float32)
  ones = jnp.ones((C, DEGW), jnp.float32)

  degp = _deg_kernel(dst, zeros128, ones)
  dinv = _dinv_call(degp)
  g1 = _mm1_call(xp, W1, dinv)
  s1 = _edge_scatter(g1, src, dst, zeros128)
  g2 = _mm2_call(s1, g1, dinv, b1.reshape(1, -1), W2)
  s2 = _edge_scatter(g2, src, dst, zeros128)
  return _fin_call(s2, g2, dinv, b2.reshape(1, -1))


# speedup vs baseline: 1.6657x; 1.5290x over previous
import functools

import jax
import jax.numpy as jnp
from jax import lax
from jax.experimental import pallas as pl
from jax.experimental.pallas import tpu as pltpu
from jax.experimental.pallas import tpu_sc as plsc

N = 10000
NPAD = 10240
E = 320000
NC = 2
NS = 16
NW = NC * NS
C = 128
CHUNKS = E // C
ITERS = -(-CHUNKS // NW)
ROWS_PER_SUB = NPAD // NS

_mesh = plsc.VectorSubcoreMesh(core_axis_name="c", subcore_axis_name="s")


def _make_edge_scatter(D):
  @functools.partial(
      pl.kernel,
      mesh=_mesh,
      out_type=jax.ShapeDtypeStruct((NC, NPAD, D), jnp.float32),
      scratch_types=[
          pltpu.VMEM((C,), jnp.int32),
          pltpu.VMEM((C,), jnp.int32),
          pltpu.VMEM((C, D), jnp.float32),
          pltpu.VMEM_SHARED((NPAD, D), jnp.float32),
          pltpu.SemaphoreType.DMA,
      ],
  )
  def scat(g_hbm, src_hbm, dst_hbm, zeros_hbm, out_hbm,
           src_v, dst_v, rows_v, acc, sem):
    cid = lax.axis_index("c")
    sid = lax.axis_index("s")
    wid = sid * NC + cid
    base = sid * ROWS_PER_SUB
    pltpu.sync_copy(zeros_hbm.at[pl.ds(base, ROWS_PER_SUB)],
                    acc.at[pl.ds(base, ROWS_PER_SUB)])
    plsc.subcore_barrier()

    def body(t, carry):
      k = wid + t * NW

      @pl.when(k < CHUNKS)
      def _():
        off = k * C
        pltpu.sync_copy(src_hbm.at[pl.ds(off, C)], src_v)
        pltpu.sync_copy(dst_hbm.at[pl.ds(off, C)], dst_v)
        pltpu.async_copy(g_hbm.at[src_v], rows_v, sem).wait()
        pltpu.sync_copy(rows_v, acc.at[dst_v], add=True)

      return carry

    lax.fori_loop(0, ITERS, body, 0)
    plsc.subcore_barrier()
    pltpu.sync_copy(acc.at[pl.ds(base, ROWS_PER_SUB)],
                    out_hbm.at[cid].at[pl.ds(base, ROWS_PER_SUB)])

  return scat


_scatter128 = _make_edge_scatter(128)

DEGW = 128


@functools.partial(
    pl.kernel,
    mesh=_mesh,
    out_type=jax.ShapeDtypeStruct((NC, NPAD, DEGW), jnp.float32),
    scratch_types=[
        pltpu.VMEM((C,), jnp.int32),
        pltpu.VMEM((C, DEGW), jnp.float32),
        pltpu.VMEM_SHARED((NPAD, DEGW), jnp.float32),
    ],
)
def _deg_kernel(dst_hbm, zeros_hbm, ones_hbm, out_hbm, dst_v, ones_v, acc):
  cid = lax.axis_index("c")
  sid = lax.axis_index("s")
  wid = sid * NC + cid
  base = sid * ROWS_PER_SUB
  pltpu.sync_copy(ones_hbm, ones_v)
  pltpu.sync_copy(zeros_hbm.at[pl.ds(base, ROWS_PER_SUB)],
                  acc.at[pl.ds(base, ROWS_PER_SUB)])
  plsc.subcore_barrier()

  def body(t, carry):
    k = wid + t * NW

    @pl.when(k < CHUNKS)
    def _():
      off = k * C
      pltpu.sync_copy(dst_hbm.at[pl.ds(off, C)], dst_v)
      pltpu.sync_copy(ones_v, acc.at[dst_v], add=True)

    return carry

  lax.fori_loop(0, ITERS, body, 0)
  plsc.subcore_barrier()
  pltpu.sync_copy(acc.at[pl.ds(base, ROWS_PER_SUB)],
                  out_hbm.at[cid].at[pl.ds(base, ROWS_PER_SUB)])


R = 1000
GRID = N // R


def _dinv_of(degp_ref):
  deg = degp_ref[0, :, 0] + degp_ref[1, :, 0] + 1.0
  return lax.rsqrt(deg)


def _mm1_body(x_ref, w_ref, degp_ref, o_ref):
  dinv = _dinv_of(degp_ref)
  h = jnp.dot(x_ref[...], w_ref[...], preferred_element_type=jnp.float32)
  o_ref[...] = h * dinv[:, None]


def _mm2_body(s_ref, g1_ref, degp_ref, b1_ref, w2_ref, o_ref):
  dinv = _dinv_of(degp_ref)
  a = (s_ref[0] + s_ref[1] + g1_ref[...]) * dinv[:, None] + b1_ref[...]
  a = jnp.maximum(a, 0.0)
  h = jnp.dot(a, w2_ref[...], preferred_element_type=jnp.float32)
  o_ref[...] = jnp.concatenate(
      [h * dinv[:, None], jnp.zeros((R, 64), jnp.float32)], axis=1)


def _fin_body(s_ref, g2_ref, degp_ref, b2_ref, o_ref):
  dinv = _dinv_of(degp_ref)
  z = ((s_ref[0, :, :64] + s_ref[1, :, :64] + g2_ref[:, :64]) * dinv[:, None]
       + b2_ref[...])
  m = jnp.max(z, axis=1, keepdims=True)
  zs = z - m
  o_ref[...] = zs - jnp.log(jnp.sum(jnp.exp(zs), axis=1, keepdims=True))


def _row_spec(width):
  return pl.BlockSpec((R, width), lambda i: (i, 0))


def _pair_spec(width):
  return pl.BlockSpec((NC, R, width), lambda i: (0, i, 0))


_degp_spec = pl.BlockSpec((NC, R, DEGW), lambda i: (0, i, 0))
_full = lambda shape: pl.BlockSpec(shape, lambda i: (0,) * len(shape))


def _mm1_call(x, W1, degp):
  return pl.pallas_call(
      _mm1_body,
      grid=(GRID,),
      in_specs=[_row_spec(128), _full((128, 128)), _degp_spec],
      out_specs=_row_spec(128),
      out_shape=jax.ShapeDtypeStruct((N, 128), jnp.float32),
  )(x, W1, degp)


def _mm2_call(s1, g1, degp, b1, W2):
  return pl.pallas_call(
      _mm2_body,
      grid=(GRID,),
      in_specs=[_pair_spec(128), _row_spec(128), _degp_spec,
                _full((1, 128)), _full((128, 64))],
      out_specs=_row_spec(128),
      out_shape=jax.ShapeDtypeStruct((N, 128), jnp.float32),
  )(s1, g1, degp, b1, W2)


def _fin_call(s2, g2, degp, b2):
  return pl.pallas_call(
      _fin_body,
      grid=(GRID,),
      in_specs=[_pair_spec(128), _row_spec(128), _degp_spec, _full((1, 64))],
      out_specs=_row_spec(64),
      out_shape=jax.ShapeDtypeStruct((N, 64), jnp.float32),
  )(s2, g2, degp, b2)


@jax.jit
def kernel(x, edge_index, W1, b1, W2, b2):
  ei = edge_index.astype(jnp.int32)
  src, dst = ei[0], ei[1]
  zeros128 = jnp.zeros((NPAD, 128), jnp.float32)
  zerosw = jnp.zeros((NPAD, DEGW), jnp.float32)
  ones = jnp.ones((C, DEGW), jnp.float32)

  degp = _deg_kernel(dst, zerosw, ones)
  g1 = _mm1_call(x, W1, degp)
  s1 = _scatter128(g1, src, dst, zeros128)
  g2 = _mm2_call(s1, g1, degp, b1.reshape(1, -1), W2)
  s2 = _scatter128(g2, src, dst, zeros128)
  return _fin_call(s2, g2, degp, b2.reshape(1, -1))
